# baseline (speedup 1.0000x reference)
import functools

import jax
import jax.numpy as jnp
from jax import lax
from jax.experimental import pallas as pl
from jax.experimental.pallas import tpu as pltpu
from jax.experimental.pallas import tpu_sc as plsc

BATCH = 4096
SEQ = 200
EMB = 64
HID = 128
OUT = 3
LANES = 16

SEQ_A = 104
SEQ_B = SEQ - SEQ_A


def _sc_gather_sum(tokens_flat, emb_bf16):
    info = plsc.get_sparse_core_info()
    nc, ns = info.num_cores, info.num_subcores
    nw = nc * ns
    rows_per_w = BATCH // nw
    tok_per_w = rows_per_w * SEQ

    mesh = plsc.VectorSubcoreMesh(core_axis_name="c", subcore_axis_name="s")

    @functools.partial(
        pl.kernel,
        out_type=jax.ShapeDtypeStruct((BATCH, EMB), jnp.float32),
        mesh=mesh,
        compiler_params=pltpu.CompilerParams(
            use_tc_tiling_on_sc=False, needs_layout_passes=False),
        scratch_types=[
            pltpu.VMEM((tok_per_w,), jnp.int32),
            pltpu.VMEM((SEQ_A, EMB), jnp.bfloat16),
            pltpu.VMEM((SEQ_B, EMB), jnp.bfloat16),
            pltpu.VMEM((SEQ_A, EMB), jnp.bfloat16),
            pltpu.VMEM((SEQ_B, EMB), jnp.bfloat16),
            pltpu.VMEM((rows_per_w, EMB), jnp.float32),
            pltpu.SemaphoreType.DMA,
            pltpu.SemaphoreType.DMA,
            pltpu.SemaphoreType.DMA,
        ],
    )
    def k(tok_hbm, table_hbm, out_hbm,
          tok_v, buf_a0, buf_b0, buf_a1, buf_b1, out_v,
          tok_sem, sem0, sem1):
        wid = lax.axis_index("s") * nc + lax.axis_index("c")
        base = wid * tok_per_w
        pltpu.async_copy(
            tok_hbm.at[pl.ds(base, tok_per_w)], tok_v, tok_sem).wait()

        bufs = ((buf_a0, buf_b0, sem0), (buf_a1, buf_b1, sem1))

        def gather_descs(r, slot):
            buf_a, buf_b, sem = bufs[slot]
            off = r * SEQ
            da = pltpu.make_async_copy(
                table_hbm.at[tok_v.at[pl.ds(off, SEQ_A)]], buf_a, sem)
            db = pltpu.make_async_copy(
                table_hbm.at[tok_v.at[pl.ds(off + SEQ_A, SEQ_B)]], buf_b, sem)
            return da, db

        def gather_start(r, slot):
            da, db = gather_descs(r, slot)
            da.start()
            db.start()

        def accum(buf, n, acc):
            def body(j, acc):
                a0, a1, a2, a3 = acc
                e0, o0 = plsc.unpack(
                    buf[j, pl.ds(0, 2 * LANES)],
                    format=plsc.PackFormat.INTERLEAVED,
                    preferred_element_type=jnp.float32)
                e1, o1 = plsc.unpack(
                    buf[j, pl.ds(2 * LANES, 2 * LANES)],
                    format=plsc.PackFormat.INTERLEAVED,
                    preferred_element_type=jnp.float32)
                return (a0 + e0, a1 + o0, a2 + e1, a3 + o1)
            return lax.fori_loop(0, n, body, acc, unroll=4)

        def process(r, slot):
            da, db = gather_descs(r, slot)
            da.wait()
            db.wait()
            buf_a, buf_b, _ = bufs[slot]
            zero = jnp.zeros((LANES,), jnp.float32)
            acc = accum(buf_a, SEQ_A, (zero, zero, zero, zero))
            acc = accum(buf_b, SEQ_B, acc)
            out_v[r, pl.ds(0, LANES)] = acc[0]
            out_v[r, pl.ds(LANES, LANES)] = acc[1]
            out_v[r, pl.ds(2 * LANES, LANES)] = acc[2]
            out_v[r, pl.ds(3 * LANES, LANES)] = acc[3]

        gather_start(0, 0)

        def outer(rr, _):
            r0 = rr * 2

            @pl.when(r0 + 1 < rows_per_w)
            def _():
                gather_start(r0 + 1, 1)
            process(r0, 0)

            @pl.when(r0 + 1 < rows_per_w)
            def _():
                @pl.when(r0 + 2 < rows_per_w)
                def _():
                    gather_start(r0 + 2, 0)
                process(r0 + 1, 1)
            return 0

        lax.fori_loop(0, rows_per_w // 2, outer, 0)
        pltpu.async_copy(
            out_v, out_hbm.at[pl.ds(wid * rows_per_w, rows_per_w)],
            tok_sem).wait()

    return k(tokens_flat, emb_bf16)


def _tc_mlp(sums, tokens, emb0, W1, b1, W2p, b2p):
    blk = 512
    grid = (BATCH // blk,)

    def body(sum_ref, tok_ref, emb0_ref, w1_ref, b1_ref, w2_ref, b2_ref,
             out_ref):
        tok = tok_ref[...]
        cnt = jnp.sum((tok != 0).astype(jnp.float32), axis=1, keepdims=True)
        n_zero = float(SEQ) - cnt
        corrected = sum_ref[...] - n_zero * emb0_ref[...]
        pooled = corrected / jnp.maximum(cnt, 1.0)
        h = jnp.dot(pooled, w1_ref[...], precision="highest") + b1_ref[...]
        h = jnp.maximum(h, 0.0)
        logits = jnp.dot(h, w2_ref[...], precision="highest") + b2_ref[...]
        m = jnp.max(logits, axis=-1, keepdims=True)
        e = jnp.exp(logits - m)
        out_ref[...] = e / jnp.sum(e, axis=-1, keepdims=True)

    return pl.pallas_call(
        body,
        grid=grid,
        in_specs=[
            pl.BlockSpec((blk, EMB), lambda i: (i, 0)),
            pl.BlockSpec((blk, SEQ), lambda i: (i, 0)),
            pl.BlockSpec((1, EMB), lambda i: (0, 0)),
            pl.BlockSpec((EMB, HID), lambda i: (0, 0)),
            pl.BlockSpec((1, HID), lambda i: (0, 0)),
            pl.BlockSpec((HID, HID), lambda i: (0, 0)),
            pl.BlockSpec((1, HID), lambda i: (0, 0)),
        ],
        out_specs=pl.BlockSpec((blk, HID), lambda i: (i, 0)),
        out_shape=jax.ShapeDtypeStruct((BATCH, HID), jnp.float32),
    )(sums, tokens, emb0, W1, b1, W2p, b2p)


_PERM = ([2 * i for i in range(16)] + [2 * i + 1 for i in range(16)]
         + [32 + 2 * i for i in range(16)] + [33 + 2 * i for i in range(16)])


def kernel(tokens, emb_table, W1, b1, W2, b2):
    tokens = tokens.astype(jnp.int32)
    emb_bf16 = emb_table.astype(jnp.bfloat16)
    sums = _sc_gather_sum(tokens.reshape(-1), emb_bf16)
    perm = jnp.asarray(_PERM, jnp.int32)
    emb0 = emb_bf16[0:1, :].astype(jnp.float32)[:, perm]
    W1p = W1[perm, :]
    W2p = jnp.pad(W2, ((0, 0), (0, HID - OUT)))
    b2p = jnp.concatenate(
        [b2, jnp.full((HID - OUT,), -1e30, jnp.float32)]).reshape(1, HID)
    out_full = _tc_mlp(sums, tokens, emb0, W1p, b1.reshape(1, HID), W2p, b2p)
    return out_full[:, :OUT]


# speedup vs baseline: 1.7248x; 1.7248x over previous
import functools

import jax
import jax.numpy as jnp
from jax import lax
from jax.experimental import pallas as pl
from jax.experimental.pallas import tpu as pltpu
from jax.experimental.pallas import tpu_sc as plsc

BATCH = 4096
SEQ = 200
EMB = 64
HID = 128
OUT = 3
LANES = 16


def _sc_gather_sum(tokens_flat, emb_table):
    info = plsc.get_sparse_core_info()
    nc, ns = info.num_cores, info.num_subcores
    nw = nc * ns
    rows_per_w = BATCH // nw
    tok_per_w = rows_per_w * SEQ

    mesh = plsc.VectorSubcoreMesh(core_axis_name="c", subcore_axis_name="s")

    @functools.partial(
        pl.kernel,
        out_type=jax.ShapeDtypeStruct((BATCH, EMB), jnp.float32),
        mesh=mesh,
        compiler_params=pltpu.CompilerParams(use_tc_tiling_on_sc=True),
        scratch_types=[
            pltpu.VMEM((tok_per_w + LANES,), jnp.int32),
            pltpu.VMEM((SEQ, EMB), jnp.float32),
            pltpu.VMEM((SEQ, EMB), jnp.float32),
            pltpu.VMEM((rows_per_w, EMB), jnp.float32),
            pltpu.SemaphoreType.DMA,
            pltpu.SemaphoreType.DMA,
            pltpu.SemaphoreType.DMA,
        ],
    )
    def k(tok_hbm, table_hbm, out_hbm,
          tok_v, buf0, buf1, out_v,
          tok_sem, sem0, sem1):
        wid = lax.axis_index("s") * nc + lax.axis_index("c")
        base = wid * tok_per_w
        pltpu.async_copy(
            tok_hbm.at[pl.ds(base, tok_per_w)],
            tok_v.at[pl.ds(0, tok_per_w)], tok_sem).wait()

        bufs = ((buf0, sem0), (buf1, sem1))

        def issue_row(r, slot):
            buf, sem = bufs[slot]
            off = r * SEQ

            def body(g, _):
                vec = tok_v[pl.ds(off + g * LANES, LANES)]
                jbase = g * LANES
                for l in range(LANES):
                    pltpu.async_copy(
                        table_hbm.at[pl.ds(vec[l], 1)],
                        buf.at[pl.ds(jbase + l, 1)], sem)
                return 0
            lax.fori_loop(0, SEQ // LANES, body, 0)
            tail = tok_v[pl.ds(off + (SEQ // LANES) * LANES, LANES)]
            for l in range(SEQ % LANES):
                pltpu.async_copy(
                    table_hbm.at[pl.ds(tail[l], 1)],
                    buf.at[pl.ds((SEQ // LANES) * LANES + l, 1)], sem)

        def drain(slot):
            buf, sem = bufs[slot]
            pltpu.make_async_copy(table_hbm.at[pl.ds(0, SEQ)], buf, sem).wait()

        def accum_store(r, slot):
            buf, _ = bufs[slot]
            zero = jnp.zeros((LANES,), jnp.float32)

            def body(j, acc):
                a0, a1, a2, a3 = acc
                a0 = a0 + buf[j, pl.ds(0, LANES)]
                a1 = a1 + buf[j, pl.ds(LANES, LANES)]
                a2 = a2 + buf[j, pl.ds(2 * LANES, LANES)]
                a3 = a3 + buf[j, pl.ds(3 * LANES, LANES)]
                return (a0, a1, a2, a3)
            acc = lax.fori_loop(0, SEQ, body, (zero, zero, zero, zero),
                                unroll=4)
            out_v[r, pl.ds(0, LANES)] = acc[0]
            out_v[r, pl.ds(LANES, LANES)] = acc[1]
            out_v[r, pl.ds(2 * LANES, LANES)] = acc[2]
            out_v[r, pl.ds(3 * LANES, LANES)] = acc[3]

        issue_row(0, 0)

        def outer(rr, _):
            r0 = rr * 2

            @pl.when(r0 + 1 < rows_per_w)
            def _():
                issue_row(r0 + 1, 1)
            drain(0)
            accum_store(r0, 0)

            @pl.when(r0 + 1 < rows_per_w)
            def _():
                @pl.when(r0 + 2 < rows_per_w)
                def _():
                    issue_row(r0 + 2, 0)
                drain(1)
                accum_store(r0 + 1, 1)
            return 0

        lax.fori_loop(0, rows_per_w // 2, outer, 0)
        pltpu.async_copy(
            out_v, out_hbm.at[pl.ds(wid * rows_per_w, rows_per_w)],
            tok_sem).wait()

    return k(tokens_flat, emb_table)


def _tc_mlp(sums, tokens, emb0, W1, b1, W2p, b2p):
    blk = 512
    grid = (BATCH // blk,)

    def body(sum_ref, tok_ref, emb0_ref, w1_ref, b1_ref, w2_ref, b2_ref,
             out_ref):
        tok = tok_ref[...]
        cnt = jnp.sum((tok != 0).astype(jnp.float32), axis=1, keepdims=True)
        n_zero = float(SEQ) - cnt
        corrected = sum_ref[...] - n_zero * emb0_ref[...]
        pooled = corrected / jnp.maximum(cnt, 1.0)
        h = jnp.dot(pooled, w1_ref[...], precision="highest") + b1_ref[...]
        h = jnp.maximum(h, 0.0)
        logits = jnp.dot(h, w2_ref[...], precision="highest") + b2_ref[...]
        m = jnp.max(logits, axis=-1, keepdims=True)
        e = jnp.exp(logits - m)
        out_ref[...] = e / jnp.sum(e, axis=-1, keepdims=True)

    return pl.pallas_call(
        body,
        grid=grid,
        in_specs=[
            pl.BlockSpec((blk, EMB), lambda i: (i, 0)),
            pl.BlockSpec((blk, SEQ), lambda i: (i, 0)),
            pl.BlockSpec((1, EMB), lambda i: (0, 0)),
            pl.BlockSpec((EMB, HID), lambda i: (0, 0)),
            pl.BlockSpec((1, HID), lambda i: (0, 0)),
            pl.BlockSpec((HID, HID), lambda i: (0, 0)),
            pl.BlockSpec((1, HID), lambda i: (0, 0)),
        ],
        out_specs=pl.BlockSpec((blk, HID), lambda i: (i, 0)),
        out_shape=jax.ShapeDtypeStruct((BATCH, HID), jnp.float32),
    )(sums, tokens, emb0, W1, b1, W2p, b2p)


def kernel(tokens, emb_table, W1, b1, W2, b2):
    tokens = tokens.astype(jnp.int32)
    sums = _sc_gather_sum(tokens.reshape(-1), emb_table)
    emb0 = emb_table[0:1, :]
    W2p = jnp.pad(W2, ((0, 0), (0, HID - OUT)))
    b2p = jnp.concatenate(
        [b2, jnp.full((HID - OUT,), -1e30, jnp.float32)]).reshape(1, HID)
    out_full = _tc_mlp(sums, tokens, emb0, W1, b1.reshape(1, HID), W2p, b2p)
    return out_full[:, :OUT]
